# Initial kernel scaffold; baseline (speedup 1.0000x reference)
import jax
import jax.numpy as jnp
from jax.experimental import pallas as pl


def kernel(z, edge_index, Wfc, bfc, W0, b0, W1, b1, W2, b2, W3, b3, W4, b4, W5, b5):
    raise NotImplementedError("write your pallas kernel here")


# speedup vs baseline: 8.5135x; 8.5135x over previous
import functools

import jax
import jax.numpy as jnp
from jax import lax
from jax.experimental import pallas as pl
from jax.experimental.pallas import tpu as pltpu
from jax.experimental.pallas import tpu_sc as plsc

N = 10000
N_PAD = 10240
LAT = 64
HID = 128
NF = 128
E = 640000
N_LAYERS = 6

NC = 2
NS = 16
EDGES_PER_SC = E // NC
EDGES_PER_TILE = EDGES_PER_SC // NS
WIN = 80
NWIN = EDGES_PER_TILE // WIN
STRIPE = N_PAD // NS

_sc_mesh = plsc.VectorSubcoreMesh(core_axis_name="c", subcore_axis_name="s")


def _zero_fill(ref, rows):
    def body(r, carry):
        for j in range(8):
            ref[r, pl.ds(j * 16, 16)] = jnp.zeros((16,), jnp.float32)
        return carry
    lax.fori_loop(0, rows, body, 0)


@functools.partial(
    pl.kernel,
    out_type=jax.ShapeDtypeStruct((NC, N_PAD), jnp.float32),
    mesh=_sc_mesh,
    scratch_types=[
        pltpu.VMEM((WIN,), jnp.int32),
        pltpu.VMEM((WIN,), jnp.float32),
        pltpu.VMEM((STRIPE,), jnp.float32),
        pltpu.VMEM_SHARED((N_PAD,), jnp.float32),
    ],
)
def _deg_sc(dst, out, idx_v, ones_v, buf_v, deg_sp):
    c = lax.axis_index("c")
    s = lax.axis_index("s")
    def fill_ones(t, carry):
        ones_v[pl.ds(t * 16, 16)] = jnp.ones((16,), jnp.float32)
        return carry
    lax.fori_loop(0, WIN // 16, fill_ones, 0)
    def fill_zero(t, carry):
        buf_v[pl.ds(t * 16, 16)] = jnp.zeros((16,), jnp.float32)
        return carry
    lax.fori_loop(0, STRIPE // 16, fill_zero, 0)
    pltpu.sync_copy(buf_v, deg_sp.at[pl.ds(s * STRIPE, STRIPE)])
    plsc.subcore_barrier()
    base = c * EDGES_PER_SC + s * EDGES_PER_TILE
    def wbody(w, carry):
        pltpu.sync_copy(dst.at[pl.ds(base + w * WIN, WIN)], idx_v)
        pltpu.sync_copy(ones_v, deg_sp.at[idx_v], add=True)
        return carry
    lax.fori_loop(0, NWIN, wbody, 0)
    plsc.subcore_barrier()
    pltpu.sync_copy(deg_sp.at[pl.ds(s * STRIPE, STRIPE)],
                    out.at[c, pl.ds(s * STRIPE, STRIPE)])


@functools.partial(
    pl.kernel,
    out_type=jax.ShapeDtypeStruct((NC, N_PAD, HID), jnp.float32),
    mesh=_sc_mesh,
    scratch_types=[
        pltpu.VMEM((WIN,), jnp.int32),
        pltpu.VMEM((WIN,), jnp.int32),
        pltpu.VMEM((WIN, HID), jnp.float32),
        pltpu.VMEM((64, HID), jnp.float32),
        pltpu.VMEM_SHARED((N_PAD, HID), jnp.float32),
        pltpu.SemaphoreType.DMA,
    ],
)
def _scatter_sc(src, dst, hh, out, src_v, dst_v, rows_v, zero_v, agg_sp, sem):
    c = lax.axis_index("c")
    s = lax.axis_index("s")
    _zero_fill(zero_v, 64)
    def zbody(k, carry):
        pltpu.sync_copy(zero_v, agg_sp.at[pl.ds(s * STRIPE + k * 64, 64), :])
        return carry
    lax.fori_loop(0, STRIPE // 64, zbody, 0)
    plsc.subcore_barrier()
    base = c * EDGES_PER_SC + s * EDGES_PER_TILE
    def wbody(w, carry):
        pltpu.sync_copy(src.at[pl.ds(base + w * WIN, WIN)], src_v)
        pltpu.sync_copy(dst.at[pl.ds(base + w * WIN, WIN)], dst_v)
        pltpu.async_copy(hh.at[src_v], rows_v, sem).wait()
        pltpu.sync_copy(rows_v, agg_sp.at[dst_v], add=True)
        return carry
    lax.fori_loop(0, NWIN, wbody, 0)
    plsc.subcore_barrier()
    pltpu.sync_copy(agg_sp.at[pl.ds(s * STRIPE, STRIPE), :],
                    out.at[c, pl.ds(s * STRIPE, STRIPE), :])


_FC_BLK = 12800
_ROW_BLK = 1000


def _fc_body(w_ref, z_ref, o_ref):
    o_ref[...] = jnp.dot(w_ref[...], z_ref[...].T,
                         preferred_element_type=jnp.float32)


def _fc(Wfc, z):
    out = pl.pallas_call(
        _fc_body,
        grid=(Wfc.shape[0] // _FC_BLK,),
        in_specs=[
            pl.BlockSpec((_FC_BLK, LAT), lambda i: (i, 0)),
            pl.BlockSpec((1, LAT), lambda i: (0, 0)),
        ],
        out_specs=pl.BlockSpec((_FC_BLK, 1), lambda i: (i, 0)),
        out_shape=jax.ShapeDtypeStruct((Wfc.shape[0], 1), jnp.float32),
    )(Wfc, z.reshape(1, LAT))
    return out.reshape(N, LAT)


def _mm0_body(x_ref, bfc_ref, d0_ref, d1_ref, w_ref, hh_ref, dinv_ref):
    dinv = lax.rsqrt(d0_ref[...] + d1_ref[...] + 1.0)
    x = jnp.maximum(x_ref[...] + bfc_ref[...], 0.0)
    h = lax.dot_general(x, w_ref[...],
                        (((1,), (1,)), ((), ())),
                        preferred_element_type=jnp.float32)
    hh_ref[...] = h * dinv
    dinv_ref[...] = dinv


def _mm0(x0, bfc2, d0, d1, W0):
    fout = W0.shape[0]
    return pl.pallas_call(
        _mm0_body,
        grid=(N // _ROW_BLK,),
        in_specs=[
            pl.BlockSpec((_ROW_BLK, LAT), lambda i: (i, 0)),
            pl.BlockSpec((_ROW_BLK, LAT), lambda i: (i, 0)),
            pl.BlockSpec((_ROW_BLK, 1), lambda i: (i, 0)),
            pl.BlockSpec((_ROW_BLK, 1), lambda i: (i, 0)),
            pl.BlockSpec((fout, LAT), lambda i: (0, 0)),
        ],
        out_specs=[
            pl.BlockSpec((_ROW_BLK, fout), lambda i: (i, 0)),
            pl.BlockSpec((_ROW_BLK, 1), lambda i: (i, 0)),
        ],
        out_shape=[
            jax.ShapeDtypeStruct((N, fout), jnp.float32),
            jax.ShapeDtypeStruct((N, 1), jnp.float32),
        ],
    )(x0, bfc2, d0, d1, W0)


def _mm_body(p0_ref, p1_ref, hh_ref, b_ref, dinv_ref, w_ref, o_ref):
    dinv = dinv_ref[...]
    x = jnp.maximum(dinv * (p0_ref[...] + p1_ref[...] + hh_ref[...])
                    + b_ref[...], 0.0)
    h = lax.dot_general(x, w_ref[...], (((1,), (1,)), ((), ())),
                        preferred_element_type=jnp.float32)
    o_ref[...] = h * dinv


def _mm(p0, p1, hh, b, dinv, W):
    fin = W.shape[1]
    fout = W.shape[0]
    return pl.pallas_call(
        _mm_body,
        grid=(N // _ROW_BLK,),
        in_specs=[
            pl.BlockSpec((_ROW_BLK, fin), lambda i: (i, 0)),
            pl.BlockSpec((_ROW_BLK, fin), lambda i: (i, 0)),
            pl.BlockSpec((_ROW_BLK, fin), lambda i: (i, 0)),
            pl.BlockSpec((1, fin), lambda i: (0, 0)),
            pl.BlockSpec((_ROW_BLK, 1), lambda i: (i, 0)),
            pl.BlockSpec((fout, fin), lambda i: (0, 0)),
        ],
        out_specs=pl.BlockSpec((_ROW_BLK, fout), lambda i: (i, 0)),
        out_shape=jax.ShapeDtypeStruct((N, fout), jnp.float32),
    )(p0, p1, hh, b.reshape(1, fin), dinv, W)


def _ep_body(p0_ref, p1_ref, hh_ref, b_ref, dinv_ref, o_ref):
    o_ref[...] = dinv_ref[...] * (p0_ref[...] + p1_ref[...] + hh_ref[...]) \
        + b_ref[...]


def _ep(p0, p1, hh, b, dinv):
    return pl.pallas_call(
        _ep_body,
        grid=(N // _ROW_BLK,),
        in_specs=[
            pl.BlockSpec((_ROW_BLK, NF), lambda i: (i, 0)),
            pl.BlockSpec((_ROW_BLK, NF), lambda i: (i, 0)),
            pl.BlockSpec((_ROW_BLK, NF), lambda i: (i, 0)),
            pl.BlockSpec((1, NF), lambda i: (0, 0)),
            pl.BlockSpec((_ROW_BLK, 1), lambda i: (i, 0)),
        ],
        out_specs=pl.BlockSpec((_ROW_BLK, NF), lambda i: (i, 0)),
        out_shape=jax.ShapeDtypeStruct((N, NF), jnp.float32),
    )(p0, p1, hh, b.reshape(1, NF), dinv)


def kernel(z, edge_index, Wfc, bfc, W0, b0, W1, b1, W2, b2, W3, b3, W4, b4, W5, b5):
    Ws = [W0, W1, W2, W3, W4, W5]
    bs = [b0, b1, b2, b3, b4, b5]
    edge = edge_index.astype(jnp.int32)
    src = edge[0]
    dst = edge[1]

    deg = _deg_sc(dst)
    x0 = _fc(Wfc, z)
    d0 = deg[0, :N].reshape(N, 1)
    d1 = deg[1, :N].reshape(N, 1)

    hh, dinv = _mm0(x0, bfc.reshape(N, LAT), d0, d1, Ws[0])
    for i in range(1, N_LAYERS):
        p = _scatter_sc(src, dst, hh)
        p0 = p[0, :N, :]
        p1 = p[1, :N, :]
        hh = _mm(p0, p1, hh, bs[i - 1], dinv, Ws[i])
    p = _scatter_sc(src, dst, hh)
    return _ep(p[0, :N, :], p[1, :N, :], hh, bs[N_LAYERS - 1], dinv)


# speedup vs baseline: 19.1616x; 2.2507x over previous
import functools

import jax
import jax.numpy as jnp
from jax import lax
from jax.experimental import pallas as pl
from jax.experimental.pallas import tpu as pltpu
from jax.experimental.pallas import tpu_sc as plsc

N = 10000
N_PAD = 10240
LAT = 64
HID = 128
NF = 128
E = 640000
N_LAYERS = 6

NC = 2
NS = 16
NT = NC * NS
EDGES_PER_TILE = E // NT
WIN = 128
WROWS = 160
PAD = WROWS * WIN - EDGES_PER_TILE
STRIPE = N_PAD // NS
NBUF = 2
NIB = 8

_sc_mesh = plsc.VectorSubcoreMesh(core_axis_name="c", subcore_axis_name="s")


def _zero_fill(ref, rows):
    def body(r, carry):
        for j in range(8):
            ref[r, pl.ds(j * 16, 16)] = jnp.zeros((16,), jnp.float32)
        return carry
    lax.fori_loop(0, rows, body, 0)


@functools.partial(
    pl.kernel,
    out_type=jax.ShapeDtypeStruct((NC, N_PAD), jnp.float32),
    mesh=_sc_mesh,
    scratch_types=[
        pltpu.VMEM((WROWS, WIN), jnp.int32),
        pltpu.VMEM((WIN,), jnp.float32),
        pltpu.VMEM((STRIPE,), jnp.float32),
        pltpu.VMEM_SHARED((N_PAD,), jnp.float32),
    ],
)
def _deg_sc(dstp, out, dst2d, ones_v, buf_v, deg_sp):
    c = lax.axis_index("c")
    s = lax.axis_index("s")
    t = c * NS + s
    def fill_ones(k, carry):
        ones_v[pl.ds(k * 16, 16)] = jnp.ones((16,), jnp.float32)
        return carry
    lax.fori_loop(0, WIN // 16, fill_ones, 0)
    def fill_zero(k, carry):
        buf_v[pl.ds(k * 16, 16)] = jnp.zeros((16,), jnp.float32)
        return carry
    lax.fori_loop(0, STRIPE // 16, fill_zero, 0)
    pltpu.sync_copy(buf_v, deg_sp.at[pl.ds(s * STRIPE, STRIPE)])
    pltpu.sync_copy(dstp.at[pl.ds(t * WROWS, WROWS), :], dst2d)
    plsc.subcore_barrier()
    def wbody(w, carry):
        pltpu.sync_copy(ones_v, deg_sp.at[dst2d.at[w]], add=True)
        return carry
    lax.fori_loop(0, WROWS, wbody, 0)
    plsc.subcore_barrier()
    pltpu.sync_copy(deg_sp.at[pl.ds(s * STRIPE, STRIPE)],
                    out.at[c, pl.ds(s * STRIPE, STRIPE)])


@functools.partial(
    pl.kernel,
    out_type=jax.ShapeDtypeStruct((NC, N_PAD, HID), jnp.float32),
    mesh=_sc_mesh,
    scratch_types=[
        pltpu.VMEM((NIB, 2, WIN), jnp.int32),
        pltpu.VMEM((NBUF, WIN, HID), jnp.float32),
        pltpu.VMEM((16, HID), jnp.float32),
        pltpu.VMEM_SHARED((N_PAD, HID), jnp.float32),
        [pltpu.SemaphoreType.DMA] * NIB,
        [pltpu.SemaphoreType.DMA] * NBUF,
        [pltpu.SemaphoreType.DMA] * NBUF,
    ],
)
def _scatter_sc(idxp, hh, out, ibuf, rows, zero_v, agg_sp, isem, gsem, ssem):
    c = lax.axis_index("c")
    s = lax.axis_index("s")
    t = c * NS + s
    _zero_fill(zero_v, 16)
    def zbody(k, carry):
        pltpu.sync_copy(zero_v, agg_sp.at[pl.ds(s * STRIPE + k * 16, 16), :])
        return carry
    lax.fori_loop(0, STRIPE // 16, zbody, 0)
    plsc.subcore_barrier()
    base = t * WROWS

    def ix_start(w, sl):
        pltpu.async_copy(idxp.at[base + w], ibuf.at[sl], isem[sl])

    def ix_wait(w, sl):
        pltpu.make_async_copy(idxp.at[base + w], ibuf.at[sl],
                              isem[sl]).wait()

    def g_start(w, sl, r):
        pltpu.async_copy(hh.at[ibuf.at[sl, 0]], rows.at[r], gsem[r])

    def g_wait(w, sl, r):
        pltpu.make_async_copy(hh.at[ibuf.at[sl, 0]], rows.at[r],
                              gsem[r]).wait()

    def s_start(w, sl, r):
        pltpu.async_copy(rows.at[r], agg_sp.at[ibuf.at[sl, 1]], ssem[r],
                         add=True)

    def s_wait(w, sl, r):
        pltpu.make_async_copy(rows.at[r], agg_sp.at[ibuf.at[sl, 1]],
                              ssem[r]).wait()

    ix_start(0, 0)
    ix_start(1, 1)
    ix_start(2, 2)
    ix_wait(0, 0)
    g_start(0, 0, 0)

    def body(k, carry):
        for j in range(NIB):
            i = k * NIB + j
            sl3 = (j + 3) % NIB
            sl1 = (j + 1) % NIB
            r1 = (j + 1) % NBUF
            r = j % NBUF
            @pl.when(i + 3 < WROWS)
            def _():
                ix_start(i + 3, sl3)
            @pl.when(i + 1 < WROWS)
            def _():
                @pl.when(i >= 1)
                def _():
                    s_wait(i - 1, (j + 7) % NIB, r1)
                ix_wait(i + 1, sl1)
                g_start(i + 1, sl1, r1)
            g_wait(i, j, r)
            s_start(i, j, r)
        return carry
    lax.fori_loop(0, WROWS // NIB, body, 0)
    s_wait(WROWS - 2, (WROWS - 2) % NIB, (WROWS - 2) % NBUF)
    s_wait(WROWS - 1, (WROWS - 1) % NIB, (WROWS - 1) % NBUF)
    plsc.subcore_barrier()
    pltpu.sync_copy(agg_sp.at[pl.ds(s * STRIPE, STRIPE), :],
                    out.at[c, pl.ds(s * STRIPE, STRIPE), :])


_FC_BLK = 12800
_ROW_BLK = 1000


def _fc_body(w_ref, z_ref, o_ref):
    o_ref[...] = jnp.dot(w_ref[...], z_ref[...].T,
                         preferred_element_type=jnp.float32)


def _fc(Wfc, z):
    out = pl.pallas_call(
        _fc_body,
        grid=(Wfc.shape[0] // _FC_BLK,),
        in_specs=[
            pl.BlockSpec((_FC_BLK, LAT), lambda i: (i, 0)),
            pl.BlockSpec((1, LAT), lambda i: (0, 0)),
        ],
        out_specs=pl.BlockSpec((_FC_BLK, 1), lambda i: (i, 0)),
        out_shape=jax.ShapeDtypeStruct((Wfc.shape[0], 1), jnp.float32),
    )(Wfc, z.reshape(1, LAT))
    return out.reshape(N, LAT)


def _mm0_body(x_ref, bfc_ref, d0_ref, d1_ref, w_ref, hh_ref, dinv_ref):
    dinv = lax.rsqrt(d0_ref[...] + d1_ref[...] + 1.0)
    x = jnp.maximum(x_ref[...] + bfc_ref[...], 0.0)
    h = lax.dot_general(x, w_ref[...],
                        (((1,), (1,)), ((), ())),
                        preferred_element_type=jnp.float32)
    hh_ref[...] = h * dinv
    dinv_ref[...] = dinv


def _mm0(x0, bfc2, d0, d1, W0):
    fout = W0.shape[0]
    return pl.pallas_call(
        _mm0_body,
        grid=(N // _ROW_BLK,),
        in_specs=[
            pl.BlockSpec((_ROW_BLK, LAT), lambda i: (i, 0)),
            pl.BlockSpec((_ROW_BLK, LAT), lambda i: (i, 0)),
            pl.BlockSpec((_ROW_BLK, 1), lambda i: (i, 0)),
            pl.BlockSpec((_ROW_BLK, 1), lambda i: (i, 0)),
            pl.BlockSpec((fout, LAT), lambda i: (0, 0)),
        ],
        out_specs=[
            pl.BlockSpec((_ROW_BLK, fout), lambda i: (i, 0)),
            pl.BlockSpec((_ROW_BLK, 1), lambda i: (i, 0)),
        ],
        out_shape=[
            jax.ShapeDtypeStruct((N, fout), jnp.float32),
            jax.ShapeDtypeStruct((N, 1), jnp.float32),
        ],
    )(x0, bfc2, d0, d1, W0)


def _mm_body(p0_ref, p1_ref, hh_ref, b_ref, dinv_ref, w_ref, o_ref):
    dinv = dinv_ref[...]
    x = jnp.maximum(dinv * (p0_ref[...] + p1_ref[...] + hh_ref[...])
                    + b_ref[...], 0.0)
    h = lax.dot_general(x, w_ref[...], (((1,), (1,)), ((), ())),
                        preferred_element_type=jnp.float32)
    o_ref[...] = h * dinv


def _mm(p0, p1, hh, b, dinv, W):
    fin = W.shape[1]
    fout = W.shape[0]
    return pl.pallas_call(
        _mm_body,
        grid=(N // _ROW_BLK,),
        in_specs=[
            pl.BlockSpec((_ROW_BLK, fin), lambda i: (i, 0)),
            pl.BlockSpec((_ROW_BLK, fin), lambda i: (i, 0)),
            pl.BlockSpec((_ROW_BLK, fin), lambda i: (i, 0)),
            pl.BlockSpec((1, fin), lambda i: (0, 0)),
            pl.BlockSpec((_ROW_BLK, 1), lambda i: (i, 0)),
            pl.BlockSpec((fout, fin), lambda i: (0, 0)),
        ],
        out_specs=pl.BlockSpec((_ROW_BLK, fout), lambda i: (i, 0)),
        out_shape=jax.ShapeDtypeStruct((N, fout), jnp.float32),
    )(p0, p1, hh, b.reshape(1, fin), dinv, W)


def _ep_body(p0_ref, p1_ref, hh_ref, b_ref, dinv_ref, o_ref):
    o_ref[...] = dinv_ref[...] * (p0_ref[...] + p1_ref[...] + hh_ref[...]) \
        + b_ref[...]


def _ep(p0, p1, hh, b, dinv):
    return pl.pallas_call(
        _ep_body,
        grid=(N // _ROW_BLK,),
        in_specs=[
            pl.BlockSpec((_ROW_BLK, NF), lambda i: (i, 0)),
            pl.BlockSpec((_ROW_BLK, NF), lambda i: (i, 0)),
            pl.BlockSpec((_ROW_BLK, NF), lambda i: (i, 0)),
            pl.BlockSpec((1, NF), lambda i: (0, 0)),
            pl.BlockSpec((_ROW_BLK, 1), lambda i: (i, 0)),
        ],
        out_specs=pl.BlockSpec((_ROW_BLK, NF), lambda i: (i, 0)),
        out_shape=jax.ShapeDtypeStruct((N, NF), jnp.float32),
    )(p0, p1, hh, b.reshape(1, NF), dinv)


def kernel(z, edge_index, Wfc, bfc, W0, b0, W1, b1, W2, b2, W3, b3, W4, b4, W5, b5):
    Ws = [W0, W1, W2, W3, W4, W5]
    bs = [b0, b1, b2, b3, b4, b5]
    edge = edge_index.astype(jnp.int32)
    src_t = edge[0].reshape(NT, EDGES_PER_TILE)
    dst_t = edge[1].reshape(NT, EDGES_PER_TILE)
    dump = N + (jnp.arange(PAD, dtype=jnp.int32) % (N_PAD - N))
    srcp = jnp.concatenate(
        [src_t, src_t[:, :PAD]], axis=1).reshape(NT * WROWS, WIN)
    dstp = jnp.concatenate(
        [dst_t, jnp.broadcast_to(dump, (NT, PAD))],
        axis=1).reshape(NT * WROWS, WIN)
    idxp = jnp.stack([srcp, dstp], axis=1)

    deg = _deg_sc(dstp)
    x0 = _fc(Wfc, z)
    d0 = deg[0, :N].reshape(N, 1)
    d1 = deg[1, :N].reshape(N, 1)

    hh, dinv = _mm0(x0, bfc.reshape(N, LAT), d0, d1, Ws[0])
    for i in range(1, N_LAYERS):
        p = _scatter_sc(idxp, hh)
        p0 = p[0, :N, :]
        p1 = p[1, :N, :]
        hh = _mm(p0, p1, hh, bs[i - 1], dinv, Ws[i])
    p = _scatter_sc(idxp, hh)
    return _ep(p[0, :N, :], p[1, :N, :], hh, bs[N_LAYERS - 1], dinv)


# speedup vs baseline: 24.8086x; 1.2947x over previous
import functools

import jax
import jax.numpy as jnp
from jax import lax
from jax.experimental import pallas as pl
from jax.experimental.pallas import tpu as pltpu
from jax.experimental.pallas import tpu_sc as plsc

N = 10000
N_PAD = 10240
LAT = 64
HID = 128
NF = 128
E = 640000
N_LAYERS = 6

NC = 2
NS = 16
NT = NC * NS
EDGES_PER_TILE = E // NT
WIN = 128
WROWS = 160
PAD = WROWS * WIN - EDGES_PER_TILE
STRIPE = N_PAD // NS
NBUF = 2
NIB = 8

_sc_mesh = plsc.VectorSubcoreMesh(core_axis_name="c", subcore_axis_name="s")


def _zero_fill(ref, rows):
    def body(r, carry):
        for j in range(8):
            ref[r, pl.ds(j * 16, 16)] = jnp.zeros((16,), jnp.float32)
        return carry
    lax.fori_loop(0, rows, body, 0)


@functools.partial(
    pl.kernel,
    out_type=jax.ShapeDtypeStruct((NC, N_PAD), jnp.float32),
    mesh=_sc_mesh,
    scratch_types=[
        pltpu.VMEM((WROWS, WIN), jnp.int32),
        pltpu.VMEM((WIN,), jnp.float32),
        pltpu.VMEM((STRIPE,), jnp.float32),
        pltpu.VMEM_SHARED((N_PAD,), jnp.float32),
    ],
)
def _deg_sc(dstp, out, dst2d, ones_v, buf_v, deg_sp):
    c = lax.axis_index("c")
    s = lax.axis_index("s")
    t = c * NS + s
    def fill_ones(k, carry):
        ones_v[pl.ds(k * 16, 16)] = jnp.ones((16,), jnp.float32)
        return carry
    lax.fori_loop(0, WIN // 16, fill_ones, 0)
    def fill_zero(k, carry):
        buf_v[pl.ds(k * 16, 16)] = jnp.zeros((16,), jnp.float32)
        return carry
    lax.fori_loop(0, STRIPE // 16, fill_zero, 0)
    pltpu.sync_copy(buf_v, deg_sp.at[pl.ds(s * STRIPE, STRIPE)])
    pltpu.sync_copy(dstp.at[pl.ds(t * WROWS, WROWS), :], dst2d)
    plsc.subcore_barrier()
    def wbody(w, carry):
        pltpu.sync_copy(ones_v, deg_sp.at[dst2d.at[w]], add=True)
        return carry
    lax.fori_loop(0, WROWS, wbody, 0)
    plsc.subcore_barrier()
    pltpu.sync_copy(deg_sp.at[pl.ds(s * STRIPE, STRIPE)],
                    out.at[c, pl.ds(s * STRIPE, STRIPE)])


@functools.partial(
    pl.kernel,
    out_type=jax.ShapeDtypeStruct((NC, N_PAD, HID), jnp.float32),
    mesh=_sc_mesh,
    scratch_types=[
        pltpu.VMEM((NIB, 2, WIN), jnp.int32),
        pltpu.VMEM((NBUF, WIN, HID), jnp.float32),
        pltpu.VMEM((16, HID), jnp.float32),
        pltpu.VMEM_SHARED((N_PAD, HID), jnp.float32),
        [pltpu.SemaphoreType.DMA] * NIB,
        [pltpu.SemaphoreType.DMA] * NBUF,
        [pltpu.SemaphoreType.DMA] * NBUF,
    ],
)
def _scatter_sc(idxp, hh, out, ibuf, rows, zero_v, agg_sp, isem, gsem, ssem):
    c = lax.axis_index("c")
    s = lax.axis_index("s")
    t = c * NS + s
    _zero_fill(zero_v, 16)
    def zbody(k, carry):
        pltpu.sync_copy(zero_v, agg_sp.at[pl.ds(s * STRIPE + k * 16, 16), :])
        return carry
    lax.fori_loop(0, STRIPE // 16, zbody, 0)
    plsc.subcore_barrier()
    base = t * WROWS

    def ix_start(w, sl):
        pltpu.async_copy(idxp.at[base + w], ibuf.at[sl], isem[sl])

    def ix_wait(w, sl):
        pltpu.make_async_copy(idxp.at[base + w], ibuf.at[sl],
                              isem[sl]).wait()

    def g_start(w, sl, r):
        pltpu.async_copy(hh.at[ibuf.at[sl, 0]], rows.at[r], gsem[r])

    def g_wait(w, sl, r):
        pltpu.make_async_copy(hh.at[ibuf.at[sl, 0]], rows.at[r],
                              gsem[r]).wait()

    def s_start(w, sl, r):
        pltpu.async_copy(rows.at[r], agg_sp.at[ibuf.at[sl, 1]], ssem[r],
                         add=True)

    def s_wait(w, sl, r):
        pltpu.make_async_copy(rows.at[r], agg_sp.at[ibuf.at[sl, 1]],
                              ssem[r]).wait()

    ix_start(0, 0)
    ix_start(1, 1)
    ix_start(2, 2)
    ix_wait(0, 0)
    g_start(0, 0, 0)

    def body(k, carry):
        for j in range(NIB):
            i = k * NIB + j
            sl3 = (j + 3) % NIB
            sl1 = (j + 1) % NIB
            r1 = (j + 1) % NBUF
            r = j % NBUF
            @pl.when(i + 3 < WROWS)
            def _():
                ix_start(i + 3, sl3)
            @pl.when(i + 1 < WROWS)
            def _():
                @pl.when(i >= 1)
                def _():
                    s_wait(i - 1, (j + 7) % NIB, r1)
                ix_wait(i + 1, sl1)
                g_start(i + 1, sl1, r1)
            g_wait(i, j, r)
            s_start(i, j, r)
        return carry
    lax.fori_loop(0, WROWS // NIB, body, 0)
    s_wait(WROWS - 2, (WROWS - 2) % NIB, (WROWS - 2) % NBUF)
    s_wait(WROWS - 1, (WROWS - 1) % NIB, (WROWS - 1) % NBUF)
    plsc.subcore_barrier()
    pltpu.sync_copy(agg_sp.at[pl.ds(s * STRIPE, STRIPE), :],
                    out.at[c, pl.ds(s * STRIPE, STRIPE), :])


_FC_BLK = 12800
_ROW_BLK = 1000


def _fc_body(z_ref, a_ref, o_ref):
    o_ref[...] = jnp.dot(z_ref[...], a_ref[...],
                         preferred_element_type=jnp.float32)


def _fc(WfcT, z):
    cols = WfcT.shape[1]
    return pl.pallas_call(
        _fc_body,
        grid=(cols // _FC_BLK,),
        in_specs=[
            pl.BlockSpec((1, LAT), lambda i: (0, 0)),
            pl.BlockSpec((LAT, _FC_BLK), lambda i: (0, i)),
        ],
        out_specs=pl.BlockSpec((1, _FC_BLK), lambda i: (0, i)),
        out_shape=jax.ShapeDtypeStruct((1, cols), jnp.float32),
    )(z.reshape(1, LAT), WfcT)


def _mm0_body(x_ref, bfc_ref, d0_ref, d1_ref, w_ref, hh_ref, dinv_ref):
    dinv = lax.rsqrt(d0_ref[...] + d1_ref[...] + 1.0)
    x = jnp.maximum(x_ref[...] + bfc_ref[...], 0.0)
    h = lax.dot_general(x, w_ref[...],
                        (((1,), (1,)), ((), ())),
                        preferred_element_type=jnp.float32)
    hh_ref[...] = h * dinv
    dinv_ref[...] = dinv


def _mm0(x0, bfc2, d0, d1, W0):
    fout = W0.shape[0]
    return pl.pallas_call(
        _mm0_body,
        grid=(N // _ROW_BLK,),
        in_specs=[
            pl.BlockSpec((_ROW_BLK, LAT), lambda i: (i, 0)),
            pl.BlockSpec((_ROW_BLK, LAT), lambda i: (i, 0)),
            pl.BlockSpec((_ROW_BLK, 1), lambda i: (i, 0)),
            pl.BlockSpec((_ROW_BLK, 1), lambda i: (i, 0)),
            pl.BlockSpec((fout, LAT), lambda i: (0, 0)),
        ],
        out_specs=[
            pl.BlockSpec((_ROW_BLK, fout), lambda i: (i, 0)),
            pl.BlockSpec((_ROW_BLK, 1), lambda i: (i, 0)),
        ],
        out_shape=[
            jax.ShapeDtypeStruct((N, fout), jnp.float32),
            jax.ShapeDtypeStruct((N, 1), jnp.float32),
        ],
    )(x0, bfc2, d0, d1, W0)


def _mm_body(p0_ref, p1_ref, hh_ref, b_ref, dinv_ref, w_ref, o_ref):
    dinv = dinv_ref[...]
    x = jnp.maximum(dinv * (p0_ref[...] + p1_ref[...] + hh_ref[...])
                    + b_ref[...], 0.0)
    h = lax.dot_general(x, w_ref[...], (((1,), (1,)), ((), ())),
                        preferred_element_type=jnp.float32)
    o_ref[...] = h * dinv


def _mm(p0, p1, hh, b, dinv, W):
    fin = W.shape[1]
    fout = W.shape[0]
    return pl.pallas_call(
        _mm_body,
        grid=(N // _ROW_BLK,),
        in_specs=[
            pl.BlockSpec((_ROW_BLK, fin), lambda i: (i, 0)),
            pl.BlockSpec((_ROW_BLK, fin), lambda i: (i, 0)),
            pl.BlockSpec((_ROW_BLK, fin), lambda i: (i, 0)),
            pl.BlockSpec((1, fin), lambda i: (0, 0)),
            pl.BlockSpec((_ROW_BLK, 1), lambda i: (i, 0)),
            pl.BlockSpec((fout, fin), lambda i: (0, 0)),
        ],
        out_specs=pl.BlockSpec((_ROW_BLK, fout), lambda i: (i, 0)),
        out_shape=jax.ShapeDtypeStruct((N, fout), jnp.float32),
    )(p0, p1, hh, b.reshape(1, fin), dinv, W)


def _ep_body(p0_ref, p1_ref, hh_ref, b_ref, dinv_ref, o_ref):
    o_ref[...] = dinv_ref[...] * (p0_ref[...] + p1_ref[...] + hh_ref[...]) \
        + b_ref[...]


def _ep(p0, p1, hh, b, dinv):
    return pl.pallas_call(
        _ep_body,
        grid=(N // _ROW_BLK,),
        in_specs=[
            pl.BlockSpec((_ROW_BLK, NF), lambda i: (i, 0)),
            pl.BlockSpec((_ROW_BLK, NF), lambda i: (i, 0)),
            pl.BlockSpec((_ROW_BLK, NF), lambda i: (i, 0)),
            pl.BlockSpec((1, NF), lambda i: (0, 0)),
            pl.BlockSpec((_ROW_BLK, 1), lambda i: (i, 0)),
        ],
        out_specs=pl.BlockSpec((_ROW_BLK, NF), lambda i: (i, 0)),
        out_shape=jax.ShapeDtypeStruct((N, NF), jnp.float32),
    )(p0, p1, hh, b.reshape(1, NF), dinv)


def kernel(z, edge_index, Wfc, bfc, W0, b0, W1, b1, W2, b2, W3, b3, W4, b4, W5, b5):
    Ws = [W0, W1, W2, W3, W4, W5]
    bs = [b0, b1, b2, b3, b4, b5]
    edge = edge_index.astype(jnp.int32)
    src_t = edge[0].reshape(NT, EDGES_PER_TILE)
    dst_t = edge[1].reshape(NT, EDGES_PER_TILE)
    dump = N + (jnp.arange(PAD, dtype=jnp.int32) % (N_PAD - N))
    srcp = jnp.concatenate(
        [src_t, src_t[:, :PAD]], axis=1).reshape(NT * WROWS, WIN)
    dstp = jnp.concatenate(
        [dst_t, jnp.broadcast_to(dump, (NT, PAD))],
        axis=1).reshape(NT * WROWS, WIN)
    idxp = jnp.stack([srcp, dstp], axis=1)

    deg = _deg_sc(dstp)
    x0 = _fc(Wfc.T, z).reshape(N, LAT)
    d0 = deg[0, :N].reshape(N, 1)
    d1 = deg[1, :N].reshape(N, 1)

    hh, dinv = _mm0(x0, bfc.reshape(N, LAT), d0, d1, Ws[0])
    for i in range(1, N_LAYERS):
        p = _scatter_sc(idxp, hh)
        p0 = p[0, :N, :]
        p1 = p[1, :N, :]
        hh = _mm(p0, p1, hh, bs[i - 1], dinv, Ws[i])
    p = _scatter_sc(idxp, hh)
    return _ep(p[0, :N, :], p[1, :N, :], hh, bs[N_LAYERS - 1], dinv)


# speedup vs baseline: 26.6011x; 1.0723x over previous
import functools

import jax
import jax.numpy as jnp
from jax import lax
from jax.experimental import pallas as pl
from jax.experimental.pallas import tpu as pltpu
from jax.experimental.pallas import tpu_sc as plsc

N = 10000
N_PAD = 10240
LAT = 64
HID = 128
NF = 128
E = 640000
N_LAYERS = 6

NC = 2
NS = 16
NT = NC * NS
WIN = 128
WROWS = 156
TILE_E = WROWS * WIN
XTRA = (E - NT * TILE_E) // WIN
STRIPE = N_PAD // NS
NBUF = 2
NIB = 4

_sc_mesh = plsc.VectorSubcoreMesh(core_axis_name="c", subcore_axis_name="s")


def _zero_fill(ref, rows):
    def body(r, carry):
        for j in range(8):
            ref[r, pl.ds(j * 16, 16)] = jnp.zeros((16,), jnp.float32)
        return carry
    lax.fori_loop(0, rows, body, 0)


@functools.partial(
    pl.kernel,
    out_type=jax.ShapeDtypeStruct((NC, N_PAD), jnp.float32),
    mesh=_sc_mesh,
    scratch_types=[
        pltpu.VMEM((NIB, 2, WIN), jnp.int32),
        pltpu.VMEM((WIN,), jnp.float32),
        pltpu.VMEM((STRIPE,), jnp.float32),
        pltpu.VMEM_SHARED((N_PAD,), jnp.float32),
        [pltpu.SemaphoreType.DMA] * NIB,
    ],
)
def _deg_sc(edge, out, ibuf, ones_v, buf_v, deg_sp, isem):
    c = lax.axis_index("c")
    s = lax.axis_index("s")
    t = c * NS + s
    base = t * TILE_E
    def fill_ones(k, carry):
        ones_v[pl.ds(k * 16, 16)] = jnp.ones((16,), jnp.float32)
        return carry
    lax.fori_loop(0, WIN // 16, fill_ones, 0)
    def fill_zero(k, carry):
        buf_v[pl.ds(k * 16, 16)] = jnp.zeros((16,), jnp.float32)
        return carry
    lax.fori_loop(0, STRIPE // 16, fill_zero, 0)
    pltpu.sync_copy(buf_v, deg_sp.at[pl.ds(s * STRIPE, STRIPE)])
    plsc.subcore_barrier()

    def ix_start(w, sl):
        pltpu.async_copy(edge.at[:, pl.ds(base + w * WIN, WIN)],
                         ibuf.at[sl], isem[sl])

    def ix_wait(w, sl):
        pltpu.make_async_copy(edge.at[:, pl.ds(base + w * WIN, WIN)],
                              ibuf.at[sl], isem[sl]).wait()

    ix_start(0, 0)
    ix_start(1, 1)
    ix_start(2, 2)

    def body(k, carry):
        for j in range(NIB):
            i = k * NIB + j
            @pl.when(i + 3 < WROWS)
            def _():
                ix_start(i + 3, (j + 3) % NIB)
            ix_wait(i, j)
            pltpu.sync_copy(ones_v, deg_sp.at[ibuf.at[j, 1]], add=True)
        return carry
    lax.fori_loop(0, WROWS // NIB, body, 0)
    @pl.when(t < XTRA)
    def _():
        pltpu.sync_copy(edge.at[:, pl.ds(NT * TILE_E + t * WIN, WIN)],
                        ibuf.at[0])
        pltpu.sync_copy(ones_v, deg_sp.at[ibuf.at[0, 1]], add=True)
    plsc.subcore_barrier()
    pltpu.sync_copy(deg_sp.at[pl.ds(s * STRIPE, STRIPE)],
                    out.at[c, pl.ds(s * STRIPE, STRIPE)])


@functools.partial(
    pl.kernel,
    out_type=jax.ShapeDtypeStruct((NC, N_PAD, HID), jnp.float32),
    mesh=_sc_mesh,
    scratch_types=[
        pltpu.VMEM((NIB, 2, WIN), jnp.int32),
        pltpu.VMEM((NBUF, WIN, HID), jnp.float32),
        pltpu.VMEM((16, HID), jnp.float32),
        pltpu.VMEM_SHARED((N_PAD, HID), jnp.float32),
        [pltpu.SemaphoreType.DMA] * NIB,
        [pltpu.SemaphoreType.DMA] * NBUF,
        [pltpu.SemaphoreType.DMA] * NBUF,
    ],
)
def _scatter_sc(edge, hh, out, ibuf, rows, zero_v, agg_sp, isem, gsem, ssem):
    c = lax.axis_index("c")
    s = lax.axis_index("s")
    t = c * NS + s
    base = t * TILE_E
    _zero_fill(zero_v, 16)
    def zbody(k, carry):
        pltpu.sync_copy(zero_v, agg_sp.at[pl.ds(s * STRIPE + k * 16, 16), :])
        return carry
    lax.fori_loop(0, STRIPE // 16, zbody, 0)
    plsc.subcore_barrier()

    def ix_start(w, sl):
        pltpu.async_copy(edge.at[:, pl.ds(base + w * WIN, WIN)],
                         ibuf.at[sl], isem[sl])

    def ix_wait(w, sl):
        pltpu.make_async_copy(edge.at[:, pl.ds(base + w * WIN, WIN)],
                              ibuf.at[sl], isem[sl]).wait()

    def g_start(sl, r):
        pltpu.async_copy(hh.at[ibuf.at[sl, 0]], rows.at[r], gsem[r])

    def g_wait(sl, r):
        pltpu.make_async_copy(hh.at[ibuf.at[sl, 0]], rows.at[r],
                              gsem[r]).wait()

    def s_start(sl, r):
        pltpu.async_copy(rows.at[r], agg_sp.at[ibuf.at[sl, 1]], ssem[r],
                         add=True)

    def s_wait(sl, r):
        pltpu.make_async_copy(rows.at[r], agg_sp.at[ibuf.at[sl, 1]],
                              ssem[r]).wait()

    ix_start(0, 0)
    ix_start(1, 1)
    ix_start(2, 2)
    ix_wait(0, 0)
    g_start(0, 0)

    def body(k, carry):
        for j in range(NIB):
            i = k * NIB + j
            sl3 = (j + 3) % NIB
            sl1 = (j + 1) % NIB
            slp = (j + 3) % NIB
            r1 = (j + 1) % NBUF
            r = j % NBUF
            @pl.when(i + 1 < WROWS)
            def _():
                @pl.when(i >= 1)
                def _():
                    s_wait(slp, r1)
            @pl.when(i + 3 < WROWS)
            def _():
                ix_start(i + 3, sl3)
            @pl.when(i + 1 < WROWS)
            def _():
                ix_wait(i + 1, sl1)
                g_start(sl1, r1)
            g_wait(j, r)
            s_start(j, r)
        return carry
    lax.fori_loop(0, WROWS // NIB, body, 0)
    s_wait((WROWS - 2) % NIB, (WROWS - 2) % NBUF)
    s_wait((WROWS - 1) % NIB, (WROWS - 1) % NBUF)
    @pl.when(t < XTRA)
    def _():
        pltpu.sync_copy(edge.at[:, pl.ds(NT * TILE_E + t * WIN, WIN)],
                        ibuf.at[0])
        pltpu.async_copy(hh.at[ibuf.at[0, 0]], rows.at[0], gsem[0]).wait()
        pltpu.sync_copy(rows.at[0], agg_sp.at[ibuf.at[0, 1]], add=True)
    plsc.subcore_barrier()
    pltpu.sync_copy(agg_sp.at[pl.ds(s * STRIPE, STRIPE), :],
                    out.at[c, pl.ds(s * STRIPE, STRIPE), :])


_FC_BLK = 12800
_ROW_BLK = 1000


def _fc_body(z_ref, a_ref, o_ref):
    o_ref[...] = jnp.dot(z_ref[...], a_ref[...],
                         preferred_element_type=jnp.float32)


def _fc(WfcT, z):
    cols = WfcT.shape[1]
    return pl.pallas_call(
        _fc_body,
        grid=(cols // _FC_BLK,),
        in_specs=[
            pl.BlockSpec((1, LAT), lambda i: (0, 0)),
            pl.BlockSpec((LAT, _FC_BLK), lambda i: (0, i)),
        ],
        out_specs=pl.BlockSpec((1, _FC_BLK), lambda i: (0, i)),
        out_shape=jax.ShapeDtypeStruct((1, cols), jnp.float32),
    )(z.reshape(1, LAT), WfcT)


def _mm0_body(x_ref, bfc_ref, d0_ref, d1_ref, w_ref, hh_ref, dinv_ref):
    dinv = lax.rsqrt(d0_ref[...] + d1_ref[...] + 1.0)
    x = jnp.maximum(x_ref[...] + bfc_ref[...], 0.0)
    h = lax.dot_general(x, w_ref[...],
                        (((1,), (1,)), ((), ())),
                        preferred_element_type=jnp.float32)
    hh_ref[...] = h * dinv
    dinv_ref[...] = dinv


def _mm0(x0, bfc2, d0, d1, W0):
    fout = W0.shape[0]
    return pl.pallas_call(
        _mm0_body,
        grid=(N // _ROW_BLK,),
        in_specs=[
            pl.BlockSpec((_ROW_BLK, LAT), lambda i: (i, 0)),
            pl.BlockSpec((_ROW_BLK, LAT), lambda i: (i, 0)),
            pl.BlockSpec((_ROW_BLK, 1), lambda i: (i, 0)),
            pl.BlockSpec((_ROW_BLK, 1), lambda i: (i, 0)),
            pl.BlockSpec((fout, LAT), lambda i: (0, 0)),
        ],
        out_specs=[
            pl.BlockSpec((_ROW_BLK, fout), lambda i: (i, 0)),
            pl.BlockSpec((_ROW_BLK, 1), lambda i: (i, 0)),
        ],
        out_shape=[
            jax.ShapeDtypeStruct((N, fout), jnp.float32),
            jax.ShapeDtypeStruct((N, 1), jnp.float32),
        ],
    )(x0, bfc2, d0, d1, W0)


def _mm_body(p0_ref, p1_ref, hh_ref, b_ref, dinv_ref, w_ref, o_ref):
    dinv = dinv_ref[...]
    x = jnp.maximum(dinv * (p0_ref[0] + p1_ref[0] + hh_ref[...])
                    + b_ref[...], 0.0)
    h = lax.dot_general(x, w_ref[...], (((1,), (1,)), ((), ())),
                        preferred_element_type=jnp.float32)
    o_ref[...] = h * dinv


def _mm(p, hh, b, dinv, W):
    fin = W.shape[1]
    fout = W.shape[0]
    return pl.pallas_call(
        _mm_body,
        grid=(N // _ROW_BLK,),
        in_specs=[
            pl.BlockSpec((1, _ROW_BLK, fin), lambda i: (0, i, 0)),
            pl.BlockSpec((1, _ROW_BLK, fin), lambda i: (1, i, 0)),
            pl.BlockSpec((_ROW_BLK, fin), lambda i: (i, 0)),
            pl.BlockSpec((1, fin), lambda i: (0, 0)),
            pl.BlockSpec((_ROW_BLK, 1), lambda i: (i, 0)),
            pl.BlockSpec((fout, fin), lambda i: (0, 0)),
        ],
        out_specs=pl.BlockSpec((_ROW_BLK, fout), lambda i: (i, 0)),
        out_shape=jax.ShapeDtypeStruct((N, fout), jnp.float32),
    )(p, p, hh, b.reshape(1, fin), dinv, W)


def _ep_body(p0_ref, p1_ref, hh_ref, b_ref, dinv_ref, o_ref):
    o_ref[...] = dinv_ref[...] * (p0_ref[0] + p1_ref[0] + hh_ref[...]) \
        + b_ref[...]


def _ep(p, hh, b, dinv):
    return pl.pallas_call(
        _ep_body,
        grid=(N // _ROW_BLK,),
        in_specs=[
            pl.BlockSpec((1, _ROW_BLK, NF), lambda i: (0, i, 0)),
            pl.BlockSpec((1, _ROW_BLK, NF), lambda i: (1, i, 0)),
            pl.BlockSpec((_ROW_BLK, NF), lambda i: (i, 0)),
            pl.BlockSpec((1, NF), lambda i: (0, 0)),
            pl.BlockSpec((_ROW_BLK, 1), lambda i: (i, 0)),
        ],
        out_specs=pl.BlockSpec((_ROW_BLK, NF), lambda i: (i, 0)),
        out_shape=jax.ShapeDtypeStruct((N, NF), jnp.float32),
    )(p, p, hh, b.reshape(1, NF), dinv)


def kernel(z, edge_index, Wfc, bfc, W0, b0, W1, b1, W2, b2, W3, b3, W4, b4, W5, b5):
    Ws = [W0, W1, W2, W3, W4, W5]
    bs = [b0, b1, b2, b3, b4, b5]
    edge = edge_index.astype(jnp.int32)

    deg = _deg_sc(edge)
    x0 = _fc(Wfc.T, z).reshape(N, LAT)
    d0 = deg[0, :N].reshape(N, 1)
    d1 = deg[1, :N].reshape(N, 1)

    hh, dinv = _mm0(x0, bfc.reshape(N, LAT), d0, d1, Ws[0])
    for i in range(1, N_LAYERS):
        p = _scatter_sc(edge, hh)
        hh = _mm(p, hh, bs[i - 1], dinv, Ws[i])
    p = _scatter_sc(edge, hh)
    return _ep(p, hh, bs[N_LAYERS - 1], dinv)


# speedup vs baseline: 27.2012x; 1.0226x over previous
import functools

import jax
import jax.numpy as jnp
from jax import lax
from jax.experimental import pallas as pl
from jax.experimental.pallas import tpu as pltpu
from jax.experimental.pallas import tpu_sc as plsc

N = 10000
N_PAD = 10240
LAT = 64
HID = 128
NF = 128
E = 640000
N_LAYERS = 6

NC = 2
NS = 16
NT = NC * NS
WIN = 128
WROWS = 156
TILE_E = WROWS * WIN
XTRA = (E - NT * TILE_E) // WIN
STRIPE = N_PAD // NS
NBUF = 2
NIB = 4

_sc_mesh = plsc.VectorSubcoreMesh(core_axis_name="c", subcore_axis_name="s")


def _zero_fill(ref, rows):
    def body(r, carry):
        for j in range(8):
            ref[r, pl.ds(j * 16, 16)] = jnp.zeros((16,), jnp.float32)
        return carry
    lax.fori_loop(0, rows, body, 0)


@functools.partial(
    pl.kernel,
    out_type=jax.ShapeDtypeStruct((NC, N_PAD), jnp.float32),
    mesh=_sc_mesh,
    scratch_types=[
        pltpu.VMEM((NIB, 2, WIN), jnp.int32),
        pltpu.VMEM((WIN,), jnp.float32),
        pltpu.VMEM((STRIPE,), jnp.float32),
        pltpu.VMEM_SHARED((N_PAD,), jnp.float32),
        [pltpu.SemaphoreType.DMA] * NIB,
    ],
)
def _deg_sc(edge, out, ibuf, ones_v, buf_v, deg_sp, isem):
    c = lax.axis_index("c")
    s = lax.axis_index("s")
    t = c * NS + s
    base = t * TILE_E
    def fill_ones(k, carry):
        ones_v[pl.ds(k * 16, 16)] = jnp.ones((16,), jnp.float32)
        return carry
    lax.fori_loop(0, WIN // 16, fill_ones, 0)
    def fill_zero(k, carry):
        buf_v[pl.ds(k * 16, 16)] = jnp.zeros((16,), jnp.float32)
        return carry
    lax.fori_loop(0, STRIPE // 16, fill_zero, 0)
    pltpu.sync_copy(buf_v, deg_sp.at[pl.ds(s * STRIPE, STRIPE)])
    plsc.subcore_barrier()

    def ix_start(w, sl):
        pltpu.async_copy(edge.at[:, pl.ds(base + w * WIN, WIN)],
                         ibuf.at[sl], isem[sl])

    def ix_wait(w, sl):
        pltpu.make_async_copy(edge.at[:, pl.ds(base + w * WIN, WIN)],
                              ibuf.at[sl], isem[sl]).wait()

    ix_start(0, 0)
    ix_start(1, 1)
    ix_start(2, 2)

    def body(k, carry):
        for j in range(NIB):
            i = k * NIB + j
            @pl.when(i + 3 < WROWS)
            def _():
                ix_start(i + 3, (j + 3) % NIB)
            ix_wait(i, j)
            pltpu.sync_copy(ones_v, deg_sp.at[ibuf.at[j, 1]], add=True)
        return carry
    lax.fori_loop(0, WROWS // NIB, body, 0)
    @pl.when(t < XTRA)
    def _():
        pltpu.sync_copy(edge.at[:, pl.ds(NT * TILE_E + t * WIN, WIN)],
                        ibuf.at[0])
        pltpu.sync_copy(ones_v, deg_sp.at[ibuf.at[0, 1]], add=True)
    plsc.subcore_barrier()
    pltpu.sync_copy(deg_sp.at[pl.ds(s * STRIPE, STRIPE)],
                    out.at[c, pl.ds(s * STRIPE, STRIPE)])


@functools.partial(
    pl.kernel,
    out_type=jax.ShapeDtypeStruct((NC, N_PAD, HID), jnp.float32),
    mesh=_sc_mesh,
    scratch_types=[
        pltpu.VMEM((NIB, 2, WIN), jnp.int32),
        pltpu.VMEM((NBUF, WIN, HID), jnp.float32),
        pltpu.VMEM((16, HID), jnp.float32),
        pltpu.VMEM_SHARED((N_PAD, HID), jnp.float32),
        [pltpu.SemaphoreType.DMA] * NIB,
        [pltpu.SemaphoreType.DMA] * NBUF,
        [pltpu.SemaphoreType.DMA] * NBUF,
    ],
)
def _scatter_sc(edge, hh, out, ibuf, rows, zero_v, agg_sp, isem, gsem, ssem):
    c = lax.axis_index("c")
    s = lax.axis_index("s")
    t = c * NS + s
    base = t * TILE_E
    _zero_fill(zero_v, 16)
    def zbody(k, carry):
        pltpu.sync_copy(zero_v, agg_sp.at[pl.ds(s * STRIPE + k * 16, 16), :])
        return carry
    lax.fori_loop(0, STRIPE // 16, zbody, 0)
    plsc.subcore_barrier()

    def ix_start(w, sl):
        pltpu.async_copy(edge.at[:, pl.ds(base + w * WIN, WIN)],
                         ibuf.at[sl], isem[sl])

    def ix_wait(w, sl):
        pltpu.make_async_copy(edge.at[:, pl.ds(base + w * WIN, WIN)],
                              ibuf.at[sl], isem[sl]).wait()

    def g_start(sl, r):
        pltpu.async_copy(hh.at[ibuf.at[sl, 0]], rows.at[r], gsem[r])

    def g_wait(sl, r):
        pltpu.make_async_copy(hh.at[ibuf.at[sl, 0]], rows.at[r],
                              gsem[r]).wait()

    def s_start(sl, r):
        pltpu.async_copy(rows.at[r], agg_sp.at[ibuf.at[sl, 1]], ssem[r],
                         add=True)

    def s_wait(sl, r):
        pltpu.make_async_copy(rows.at[r], agg_sp.at[ibuf.at[sl, 1]],
                              ssem[r]).wait()

    ix_start(0, 0)
    ix_start(1, 1)
    ix_start(2, 2)
    ix_wait(0, 0)
    g_start(0, 0)

    def body(k, carry):
        for j in range(NIB):
            i = k * NIB + j
            sl3 = (j + 3) % NIB
            sl1 = (j + 1) % NIB
            slp = (j + 3) % NIB
            r1 = (j + 1) % NBUF
            r = j % NBUF
            @pl.when(i + 1 < WROWS)
            def _():
                @pl.when(i >= 1)
                def _():
                    s_wait(slp, r1)
            @pl.when(i + 3 < WROWS)
            def _():
                ix_start(i + 3, sl3)
            @pl.when(i + 1 < WROWS)
            def _():
                ix_wait(i + 1, sl1)
                g_start(sl1, r1)
            g_wait(j, r)
            s_start(j, r)
        return carry
    lax.fori_loop(0, WROWS // NIB, body, 0)
    s_wait((WROWS - 2) % NIB, (WROWS - 2) % NBUF)
    s_wait((WROWS - 1) % NIB, (WROWS - 1) % NBUF)
    @pl.when(t < XTRA)
    def _():
        pltpu.sync_copy(edge.at[:, pl.ds(NT * TILE_E + t * WIN, WIN)],
                        ibuf.at[0])
        pltpu.async_copy(hh.at[ibuf.at[0, 0]], rows.at[0], gsem[0]).wait()
        pltpu.sync_copy(rows.at[0], agg_sp.at[ibuf.at[0, 1]], add=True)
    plsc.subcore_barrier()
    pltpu.sync_copy(agg_sp.at[pl.ds(s * STRIPE, STRIPE), :],
                    out.at[c, pl.ds(s * STRIPE, STRIPE), :])


_FC_BLK = 25600
_ROW_BLK = 2000


def _fc_body(z_ref, a_ref, o_ref):
    o_ref[...] = jnp.dot(z_ref[...], a_ref[...],
                         preferred_element_type=jnp.float32)


def _fc(WfcT, z):
    cols = WfcT.shape[1]
    return pl.pallas_call(
        _fc_body,
        grid=(cols // _FC_BLK,),
        in_specs=[
            pl.BlockSpec((1, LAT), lambda i: (0, 0)),
            pl.BlockSpec((LAT, _FC_BLK), lambda i: (0, i)),
        ],
        out_specs=pl.BlockSpec((1, _FC_BLK), lambda i: (0, i)),
        out_shape=jax.ShapeDtypeStruct((1, cols), jnp.float32),
    )(z.reshape(1, LAT), WfcT)


def _mm0_body(x_ref, bfc_ref, d0_ref, d1_ref, w_ref, hh_ref, dinv_ref):
    dinv = lax.rsqrt(d0_ref[...] + d1_ref[...] + 1.0)
    x = jnp.maximum(x_ref[...] + bfc_ref[...], 0.0)
    h = lax.dot_general(x, w_ref[...],
                        (((1,), (1,)), ((), ())),
                        preferred_element_type=jnp.float32)
    hh_ref[...] = h * dinv
    dinv_ref[...] = dinv


def _mm0(x0, bfc2, d0, d1, W0):
    fout = W0.shape[0]
    return pl.pallas_call(
        _mm0_body,
        grid=(N // _ROW_BLK,),
        in_specs=[
            pl.BlockSpec((_ROW_BLK, LAT), lambda i: (i, 0)),
            pl.BlockSpec((_ROW_BLK, LAT), lambda i: (i, 0)),
            pl.BlockSpec((_ROW_BLK, 1), lambda i: (i, 0)),
            pl.BlockSpec((_ROW_BLK, 1), lambda i: (i, 0)),
            pl.BlockSpec((fout, LAT), lambda i: (0, 0)),
        ],
        out_specs=[
            pl.BlockSpec((_ROW_BLK, fout), lambda i: (i, 0)),
            pl.BlockSpec((_ROW_BLK, 1), lambda i: (i, 0)),
        ],
        out_shape=[
            jax.ShapeDtypeStruct((N, fout), jnp.float32),
            jax.ShapeDtypeStruct((N, 1), jnp.float32),
        ],
    )(x0, bfc2, d0, d1, W0)


def _mm_body(p0_ref, p1_ref, hh_ref, b_ref, dinv_ref, w_ref, o_ref):
    dinv = dinv_ref[...]
    x = jnp.maximum(dinv * (p0_ref[0] + p1_ref[0] + hh_ref[...])
                    + b_ref[...], 0.0)
    h = lax.dot_general(x, w_ref[...], (((1,), (1,)), ((), ())),
                        preferred_element_type=jnp.float32)
    o_ref[...] = h * dinv


def _mm(p, hh, b, dinv, W):
    fin = W.shape[1]
    fout = W.shape[0]
    return pl.pallas_call(
        _mm_body,
        grid=(N // _ROW_BLK,),
        in_specs=[
            pl.BlockSpec((1, _ROW_BLK, fin), lambda i: (0, i, 0)),
            pl.BlockSpec((1, _ROW_BLK, fin), lambda i: (1, i, 0)),
            pl.BlockSpec((_ROW_BLK, fin), lambda i: (i, 0)),
            pl.BlockSpec((1, fin), lambda i: (0, 0)),
            pl.BlockSpec((_ROW_BLK, 1), lambda i: (i, 0)),
            pl.BlockSpec((fout, fin), lambda i: (0, 0)),
        ],
        out_specs=pl.BlockSpec((_ROW_BLK, fout), lambda i: (i, 0)),
        out_shape=jax.ShapeDtypeStruct((N, fout), jnp.float32),
    )(p, p, hh, b.reshape(1, fin), dinv, W)


def _ep_body(p0_ref, p1_ref, hh_ref, b_ref, dinv_ref, o_ref):
    o_ref[...] = dinv_ref[...] * (p0_ref[0] + p1_ref[0] + hh_ref[...]) \
        + b_ref[...]


def _ep(p, hh, b, dinv):
    return pl.pallas_call(
        _ep_body,
        grid=(N // _ROW_BLK,),
        in_specs=[
            pl.BlockSpec((1, _ROW_BLK, NF), lambda i: (0, i, 0)),
            pl.BlockSpec((1, _ROW_BLK, NF), lambda i: (1, i, 0)),
            pl.BlockSpec((_ROW_BLK, NF), lambda i: (i, 0)),
            pl.BlockSpec((1, NF), lambda i: (0, 0)),
            pl.BlockSpec((_ROW_BLK, 1), lambda i: (i, 0)),
        ],
        out_specs=pl.BlockSpec((_ROW_BLK, NF), lambda i: (i, 0)),
        out_shape=jax.ShapeDtypeStruct((N, NF), jnp.float32),
    )(p, p, hh, b.reshape(1, NF), dinv)


def kernel(z, edge_index, Wfc, bfc, W0, b0, W1, b1, W2, b2, W3, b3, W4, b4, W5, b5):
    Ws = [W0, W1, W2, W3, W4, W5]
    bs = [b0, b1, b2, b3, b4, b5]
    edge = edge_index.astype(jnp.int32)

    deg = _deg_sc(edge)
    x0 = _fc(Wfc.T, z).reshape(N, LAT)
    d0 = deg[0, :N].reshape(N, 1)
    d1 = deg[1, :N].reshape(N, 1)

    hh, dinv = _mm0(x0, bfc.reshape(N, LAT), d0, d1, Ws[0])
    for i in range(1, N_LAYERS):
        p = _scatter_sc(edge, hh)
        hh = _mm(p, hh, bs[i - 1], dinv, Ws[i])
    p = _scatter_sc(edge, hh)
    return _ep(p, hh, bs[N_LAYERS - 1], dinv)


# speedup vs baseline: 27.6094x; 1.0150x over previous
import functools

import jax
import jax.numpy as jnp
from jax import lax
from jax.experimental import pallas as pl
from jax.experimental.pallas import tpu as pltpu
from jax.experimental.pallas import tpu_sc as plsc

N = 10000
N_PAD = 10240
LAT = 64
HID = 128
NF = 128
E = 640000
N_LAYERS = 6

NC = 2
NS = 16
NT = NC * NS
WIN = 128
WROWS = 156
TILE_E = WROWS * WIN
XTRA = (E - NT * TILE_E) // WIN
STRIPE = N_PAD // NS
NBUF = 2
NIB = 4

_sc_mesh = plsc.VectorSubcoreMesh(core_axis_name="c", subcore_axis_name="s")


def _zero_fill(ref, rows):
    def body(r, carry):
        for j in range(8):
            ref[r, pl.ds(j * 16, 16)] = jnp.zeros((16,), jnp.float32)
        return carry
    lax.fori_loop(0, rows, body, 0)


@functools.partial(
    pl.kernel,
    out_type=jax.ShapeDtypeStruct((NC, N_PAD), jnp.float32),
    mesh=_sc_mesh,
    scratch_types=[
        pltpu.VMEM((NIB, 2, WIN), jnp.int32),
        pltpu.VMEM((WIN,), jnp.float32),
        pltpu.VMEM((STRIPE,), jnp.float32),
        pltpu.VMEM_SHARED((N_PAD,), jnp.float32),
        [pltpu.SemaphoreType.DMA] * NIB,
    ],
)
def _deg_sc(edge, out, ibuf, ones_v, buf_v, deg_sp, isem):
    c = lax.axis_index("c")
    s = lax.axis_index("s")
    t = c * NS + s
    base = t * TILE_E
    def fill_ones(k, carry):
        ones_v[pl.ds(k * 16, 16)] = jnp.ones((16,), jnp.float32)
        return carry
    lax.fori_loop(0, WIN // 16, fill_ones, 0)
    def fill_zero(k, carry):
        buf_v[pl.ds(k * 16, 16)] = jnp.zeros((16,), jnp.float32)
        return carry
    lax.fori_loop(0, STRIPE // 16, fill_zero, 0)
    pltpu.sync_copy(buf_v, deg_sp.at[pl.ds(s * STRIPE, STRIPE)])
    plsc.subcore_barrier()

    def ix_start(w, sl):
        pltpu.async_copy(edge.at[:, pl.ds(base + w * WIN, WIN)],
                         ibuf.at[sl], isem[sl])

    def ix_wait(w, sl):
        pltpu.make_async_copy(edge.at[:, pl.ds(base + w * WIN, WIN)],
                              ibuf.at[sl], isem[sl]).wait()

    ix_start(0, 0)
    ix_start(1, 1)
    ix_start(2, 2)

    def body(k, carry):
        for j in range(NIB):
            i = k * NIB + j
            @pl.when(i + 3 < WROWS)
            def _():
                ix_start(i + 3, (j + 3) % NIB)
            ix_wait(i, j)
            pltpu.sync_copy(ones_v, deg_sp.at[ibuf.at[j, 1]], add=True)
        return carry
    lax.fori_loop(0, WROWS // NIB, body, 0)
    @pl.when(t < XTRA)
    def _():
        pltpu.sync_copy(edge.at[:, pl.ds(NT * TILE_E + t * WIN, WIN)],
                        ibuf.at[0])
        pltpu.sync_copy(ones_v, deg_sp.at[ibuf.at[0, 1]], add=True)
    plsc.subcore_barrier()
    pltpu.sync_copy(deg_sp.at[pl.ds(s * STRIPE, STRIPE)],
                    out.at[c, pl.ds(s * STRIPE, STRIPE)])


@functools.partial(
    pl.kernel,
    out_type=jax.ShapeDtypeStruct((NC, N_PAD, HID), jnp.float32),
    mesh=_sc_mesh,
    scratch_types=[
        pltpu.VMEM((NIB, 2, WIN), jnp.int32),
        pltpu.VMEM((NBUF, WIN, HID), jnp.float32),
        pltpu.VMEM((16, HID), jnp.float32),
        pltpu.VMEM_SHARED((N_PAD, HID), jnp.float32),
        [pltpu.SemaphoreType.DMA] * NIB,
        [pltpu.SemaphoreType.DMA] * NBUF,
        [pltpu.SemaphoreType.DMA] * NBUF,
    ],
)
def _scatter_sc(edge, hh, out, ibuf, rows, zero_v, agg_sp, isem, gsem, ssem):
    c = lax.axis_index("c")
    s = lax.axis_index("s")
    t = c * NS + s
    base = t * TILE_E
    _zero_fill(zero_v, 16)
    def zbody(k, carry):
        pltpu.sync_copy(zero_v, agg_sp.at[pl.ds(s * STRIPE + k * 16, 16), :])
        return carry
    lax.fori_loop(0, STRIPE // 16, zbody, 0)
    plsc.subcore_barrier()

    def ix_start(w, sl):
        pltpu.async_copy(edge.at[:, pl.ds(base + w * WIN, WIN)],
                         ibuf.at[sl], isem[sl])

    def ix_wait(w, sl):
        pltpu.make_async_copy(edge.at[:, pl.ds(base + w * WIN, WIN)],
                              ibuf.at[sl], isem[sl]).wait()

    def g_start(sl, r):
        pltpu.async_copy(hh.at[ibuf.at[sl, 0]], rows.at[r], gsem[r])

    def g_wait(sl, r):
        pltpu.make_async_copy(hh.at[ibuf.at[sl, 0]], rows.at[r],
                              gsem[r]).wait()

    def s_start(sl, r):
        pltpu.async_copy(rows.at[r], agg_sp.at[ibuf.at[sl, 1]], ssem[r],
                         add=True)

    def s_wait(sl, r):
        pltpu.make_async_copy(rows.at[r], agg_sp.at[ibuf.at[sl, 1]],
                              ssem[r]).wait()

    ix_start(0, 0)
    ix_start(1, 1)
    ix_start(2, 2)
    ix_wait(0, 0)
    g_start(0, 0)

    def body(k, carry):
        for j in range(NIB):
            i = k * NIB + j
            sl3 = (j + 3) % NIB
            sl1 = (j + 1) % NIB
            slp = (j + 3) % NIB
            r1 = (j + 1) % NBUF
            r = j % NBUF
            @pl.when(i + 1 < WROWS)
            def _():
                @pl.when(i >= 1)
                def _():
                    s_wait(slp, r1)
            @pl.when(i + 3 < WROWS)
            def _():
                ix_start(i + 3, sl3)
            @pl.when(i + 1 < WROWS)
            def _():
                ix_wait(i + 1, sl1)
                g_start(sl1, r1)
            g_wait(j, r)
            s_start(j, r)
        return carry
    lax.fori_loop(0, WROWS // NIB, body, 0)
    s_wait((WROWS - 2) % NIB, (WROWS - 2) % NBUF)
    s_wait((WROWS - 1) % NIB, (WROWS - 1) % NBUF)
    @pl.when(t < XTRA)
    def _():
        pltpu.sync_copy(edge.at[:, pl.ds(NT * TILE_E + t * WIN, WIN)],
                        ibuf.at[0])
        pltpu.async_copy(hh.at[ibuf.at[0, 0]], rows.at[0], gsem[0]).wait()
        pltpu.sync_copy(rows.at[0], agg_sp.at[ibuf.at[0, 1]], add=True)
    plsc.subcore_barrier()
    pltpu.sync_copy(agg_sp.at[pl.ds(s * STRIPE, STRIPE), :],
                    out.at[c, pl.ds(s * STRIPE, STRIPE), :])


_FC_BLK = 16000
_ROW_BLK = 2000


def _fc(WfcT, z):
    cols = WfcT.shape[1]
    rcols = cols // 8
    cblk = 3200
    kblk = rcols // cblk

    def body(z_ref, *refs):
        o_ref = refs[8]
        zz = z_ref[...]
        rows = [jnp.dot(zz, refs[r][...], preferred_element_type=jnp.float32)
                for r in range(8)]
        o_ref[...] = jnp.concatenate(rows, axis=0)

    def make_spec(r):
        return pl.BlockSpec((LAT, cblk), lambda k, rr=r: (0, rr * kblk + k))

    return pl.pallas_call(
        body,
        grid=(kblk,),
        in_specs=[pl.BlockSpec((1, LAT), lambda k: (0, 0))]
        + [make_spec(r) for r in range(8)],
        out_specs=pl.BlockSpec((8, cblk), lambda k: (0, k)),
        out_shape=jax.ShapeDtypeStruct((8, rcols), jnp.float32),
    )(z.reshape(1, LAT), *([WfcT] * 8))


def _mm0_body(x_ref, bfc_ref, d0_ref, d1_ref, w_ref, hh_ref, dinv_ref):
    dinv = lax.rsqrt(d0_ref[...] + d1_ref[...] + 1.0)
    x = jnp.maximum(x_ref[...] + bfc_ref[...], 0.0)
    h = lax.dot_general(x, w_ref[...],
                        (((1,), (1,)), ((), ())),
                        preferred_element_type=jnp.float32)
    hh_ref[...] = h * dinv
    dinv_ref[...] = dinv


def _mm0(x0, bfc2, d0, d1, W0):
    fout = W0.shape[0]
    return pl.pallas_call(
        _mm0_body,
        grid=(N // _ROW_BLK,),
        in_specs=[
            pl.BlockSpec((_ROW_BLK, LAT), lambda i: (i, 0)),
            pl.BlockSpec((_ROW_BLK, LAT), lambda i: (i, 0)),
            pl.BlockSpec((_ROW_BLK, 1), lambda i: (i, 0)),
            pl.BlockSpec((_ROW_BLK, 1), lambda i: (i, 0)),
            pl.BlockSpec((fout, LAT), lambda i: (0, 0)),
        ],
        out_specs=[
            pl.BlockSpec((_ROW_BLK, fout), lambda i: (i, 0)),
            pl.BlockSpec((_ROW_BLK, 1), lambda i: (i, 0)),
        ],
        out_shape=[
            jax.ShapeDtypeStruct((N, fout), jnp.float32),
            jax.ShapeDtypeStruct((N, 1), jnp.float32),
        ],
    )(x0, bfc2, d0, d1, W0)


def _mm_body(p0_ref, p1_ref, hh_ref, b_ref, dinv_ref, w_ref, o_ref):
    dinv = dinv_ref[...]
    x = jnp.maximum(dinv * (p0_ref[0] + p1_ref[0] + hh_ref[...])
                    + b_ref[...], 0.0)
    h = lax.dot_general(x, w_ref[...], (((1,), (1,)), ((), ())),
                        preferred_element_type=jnp.float32)
    o_ref[...] = h * dinv


def _mm(p, hh, b, dinv, W):
    fin = W.shape[1]
    fout = W.shape[0]
    return pl.pallas_call(
        _mm_body,
        grid=(N // _ROW_BLK,),
        in_specs=[
            pl.BlockSpec((1, _ROW_BLK, fin), lambda i: (0, i, 0)),
            pl.BlockSpec((1, _ROW_BLK, fin), lambda i: (1, i, 0)),
            pl.BlockSpec((_ROW_BLK, fin), lambda i: (i, 0)),
            pl.BlockSpec((1, fin), lambda i: (0, 0)),
            pl.BlockSpec((_ROW_BLK, 1), lambda i: (i, 0)),
            pl.BlockSpec((fout, fin), lambda i: (0, 0)),
        ],
        out_specs=pl.BlockSpec((_ROW_BLK, fout), lambda i: (i, 0)),
        out_shape=jax.ShapeDtypeStruct((N, fout), jnp.float32),
    )(p, p, hh, b.reshape(1, fin), dinv, W)


def _ep_body(p0_ref, p1_ref, hh_ref, b_ref, dinv_ref, o_ref):
    o_ref[...] = dinv_ref[...] * (p0_ref[0] + p1_ref[0] + hh_ref[...]) \
        + b_ref[...]


def _ep(p, hh, b, dinv):
    return pl.pallas_call(
        _ep_body,
        grid=(N // _ROW_BLK,),
        in_specs=[
            pl.BlockSpec((1, _ROW_BLK, NF), lambda i: (0, i, 0)),
            pl.BlockSpec((1, _ROW_BLK, NF), lambda i: (1, i, 0)),
            pl.BlockSpec((_ROW_BLK, NF), lambda i: (i, 0)),
            pl.BlockSpec((1, NF), lambda i: (0, 0)),
            pl.BlockSpec((_ROW_BLK, 1), lambda i: (i, 0)),
        ],
        out_specs=pl.BlockSpec((_ROW_BLK, NF), lambda i: (i, 0)),
        out_shape=jax.ShapeDtypeStruct((N, NF), jnp.float32),
    )(p, p, hh, b.reshape(1, NF), dinv)


def kernel(z, edge_index, Wfc, bfc, W0, b0, W1, b1, W2, b2, W3, b3, W4, b4, W5, b5):
    Ws = [W0, W1, W2, W3, W4, W5]
    bs = [b0, b1, b2, b3, b4, b5]
    edge = edge_index.astype(jnp.int32)

    deg = _deg_sc(edge)
    x0 = _fc(Wfc.T, z).reshape(N, LAT)
    d0 = deg[0, :N].reshape(N, 1)
    d1 = deg[1, :N].reshape(N, 1)

    hh, dinv = _mm0(x0, bfc.reshape(N, LAT), d0, d1, Ws[0])
    for i in range(1, N_LAYERS):
        p = _scatter_sc(edge, hh)
        hh = _mm(p, hh, bs[i - 1], dinv, Ws[i])
    p = _scatter_sc(edge, hh)
    return _ep(p, hh, bs[N_LAYERS - 1], dinv)


# speedup vs baseline: 27.8124x; 1.0074x over previous
import functools

import jax
import jax.numpy as jnp
from jax import lax
from jax.experimental import pallas as pl
from jax.experimental.pallas import tpu as pltpu
from jax.experimental.pallas import tpu_sc as plsc

N = 10000
N_PAD = 10240
LAT = 64
HID = 128
NF = 128
E = 640000
N_LAYERS = 6

NC = 2
NS = 16
NT = NC * NS
WIN = 128
WROWS = 156
TILE_E = WROWS * WIN
XTRA = (E - NT * TILE_E) // WIN
STRIPE = N_PAD // NS
NBUF = 2
NIB = 4

_sc_mesh = plsc.VectorSubcoreMesh(core_axis_name="c", subcore_axis_name="s")


def _zero_fill(ref, rows):
    def body(r, carry):
        for j in range(8):
            ref[r, pl.ds(j * 16, 16)] = jnp.zeros((16,), jnp.float32)
        return carry
    lax.fori_loop(0, rows, body, 0)


@functools.partial(
    pl.kernel,
    out_type=jax.ShapeDtypeStruct((NC, N_PAD), jnp.float32),
    mesh=_sc_mesh,
    scratch_types=[
        pltpu.VMEM((NIB, 2, WIN), jnp.int32),
        pltpu.VMEM((WIN,), jnp.float32),
        pltpu.VMEM((STRIPE,), jnp.float32),
        pltpu.VMEM_SHARED((N_PAD,), jnp.float32),
        [pltpu.SemaphoreType.DMA] * NIB,
    ],
)
def _deg_sc(edge, out, ibuf, ones_v, buf_v, deg_sp, isem):
    c = lax.axis_index("c")
    s = lax.axis_index("s")
    t = c * NS + s
    base = t * TILE_E
    def fill_ones(k, carry):
        ones_v[pl.ds(k * 16, 16)] = jnp.ones((16,), jnp.float32)
        return carry
    lax.fori_loop(0, WIN // 16, fill_ones, 0)
    def fill_zero(k, carry):
        buf_v[pl.ds(k * 16, 16)] = jnp.zeros((16,), jnp.float32)
        return carry
    lax.fori_loop(0, STRIPE // 16, fill_zero, 0)
    pltpu.sync_copy(buf_v, deg_sp.at[pl.ds(s * STRIPE, STRIPE)])
    plsc.subcore_barrier()

    def ix_start(w, sl):
        pltpu.async_copy(edge.at[:, pl.ds(base + w * WIN, WIN)],
                         ibuf.at[sl], isem[sl])

    def ix_wait(w, sl):
        pltpu.make_async_copy(edge.at[:, pl.ds(base + w * WIN, WIN)],
                              ibuf.at[sl], isem[sl]).wait()

    ix_start(0, 0)
    ix_start(1, 1)
    ix_start(2, 2)

    def body(k, carry):
        for j in range(NIB):
            i = k * NIB + j
            @pl.when(i + 3 < WROWS)
            def _():
                ix_start(i + 3, (j + 3) % NIB)
            ix_wait(i, j)
            pltpu.sync_copy(ones_v, deg_sp.at[ibuf.at[j, 1]], add=True)
        return carry
    lax.fori_loop(0, WROWS // NIB, body, 0)
    @pl.when(t < XTRA)
    def _():
        pltpu.sync_copy(edge.at[:, pl.ds(NT * TILE_E + t * WIN, WIN)],
                        ibuf.at[0])
        pltpu.sync_copy(ones_v, deg_sp.at[ibuf.at[0, 1]], add=True)
    plsc.subcore_barrier()
    pltpu.sync_copy(deg_sp.at[pl.ds(s * STRIPE, STRIPE)],
                    out.at[c, pl.ds(s * STRIPE, STRIPE)])


@functools.partial(
    pl.kernel,
    out_type=jax.ShapeDtypeStruct((NC, N_PAD, HID), jnp.float32),
    mesh=_sc_mesh,
    scratch_types=[
        pltpu.VMEM((NIB, 2, WIN), jnp.int32),
        pltpu.VMEM((NBUF, WIN, HID), jnp.float32),
        pltpu.VMEM((16, HID), jnp.float32),
        pltpu.VMEM_SHARED((N_PAD, HID), jnp.float32),
        [pltpu.SemaphoreType.DMA] * NIB,
        [pltpu.SemaphoreType.DMA] * NBUF,
        [pltpu.SemaphoreType.DMA] * NBUF,
    ],
)
def _scatter_sc(edge, hh, out, ibuf, rows, zero_v, agg_sp, isem, gsem, ssem):
    c = lax.axis_index("c")
    s = lax.axis_index("s")
    t = c * NS + s
    base = t * TILE_E

    def ix_start(w, sl):
        pltpu.async_copy(edge.at[:, pl.ds(base + w * WIN, WIN)],
                         ibuf.at[sl], isem[sl])

    def ix_wait(w, sl):
        pltpu.make_async_copy(edge.at[:, pl.ds(base + w * WIN, WIN)],
                              ibuf.at[sl], isem[sl]).wait()

    def g_start(sl, r):
        pltpu.async_copy(hh.at[ibuf.at[sl, 0]], rows.at[r], gsem[r])

    def g_wait(sl, r):
        pltpu.make_async_copy(hh.at[ibuf.at[sl, 0]], rows.at[r],
                              gsem[r]).wait()

    def s_start(sl, r):
        pltpu.async_copy(rows.at[r], agg_sp.at[ibuf.at[sl, 1]], ssem[r],
                         add=True)

    def s_wait(sl, r):
        pltpu.make_async_copy(rows.at[r], agg_sp.at[ibuf.at[sl, 1]],
                              ssem[r]).wait()

    ix_start(0, 0)
    ix_start(1, 1)
    ix_start(2, 2)
    ix_wait(0, 0)
    g_start(0, 0)

    _zero_fill(zero_v, 16)
    def zbody(k, carry):
        pltpu.sync_copy(zero_v, agg_sp.at[pl.ds(s * STRIPE + k * 16, 16), :])
        return carry
    lax.fori_loop(0, STRIPE // 16, zbody, 0)
    plsc.subcore_barrier()

    def body(k, carry):
        for j in range(NIB):
            i = k * NIB + j
            sl3 = (j + 3) % NIB
            sl1 = (j + 1) % NIB
            slp = (j + 3) % NIB
            r1 = (j + 1) % NBUF
            r = j % NBUF
            @pl.when(i + 1 < WROWS)
            def _():
                @pl.when(i >= 1)
                def _():
                    s_wait(slp, r1)
            @pl.when(i + 3 < WROWS)
            def _():
                ix_start(i + 3, sl3)
            @pl.when(i + 1 < WROWS)
            def _():
                ix_wait(i + 1, sl1)
                g_start(sl1, r1)
            g_wait(j, r)
            s_start(j, r)
        return carry
    lax.fori_loop(0, WROWS // NIB, body, 0)
    s_wait((WROWS - 2) % NIB, (WROWS - 2) % NBUF)
    s_wait((WROWS - 1) % NIB, (WROWS - 1) % NBUF)
    @pl.when(t < XTRA)
    def _():
        pltpu.sync_copy(edge.at[:, pl.ds(NT * TILE_E + t * WIN, WIN)],
                        ibuf.at[0])
        pltpu.async_copy(hh.at[ibuf.at[0, 0]], rows.at[0], gsem[0]).wait()
        pltpu.sync_copy(rows.at[0], agg_sp.at[ibuf.at[0, 1]], add=True)
    plsc.subcore_barrier()
    pltpu.sync_copy(agg_sp.at[pl.ds(s * STRIPE, STRIPE), :],
                    out.at[c, pl.ds(s * STRIPE, STRIPE), :])


_FC_BLK = 16000
_ROW_BLK = 2000


def _fc(WfcT, z):
    cols = WfcT.shape[1]
    rcols = cols // 8
    cblk = 3200
    kblk = rcols // cblk

    def body(z_ref, *refs):
        o_ref = refs[8]
        zz = z_ref[...]
        rows = [jnp.dot(zz, refs[r][...], preferred_element_type=jnp.float32)
                for r in range(8)]
        o_ref[...] = jnp.concatenate(rows, axis=0)

    def make_spec(r):
        return pl.BlockSpec((LAT, cblk), lambda k, rr=r: (0, rr * kblk + k))

    return pl.pallas_call(
        body,
        grid=(kblk,),
        in_specs=[pl.BlockSpec((1, LAT), lambda k: (0, 0))]
        + [make_spec(r) for r in range(8)],
        out_specs=pl.BlockSpec((8, cblk), lambda k: (0, k)),
        out_shape=jax.ShapeDtypeStruct((8, rcols), jnp.float32),
    )(z.reshape(1, LAT), *([WfcT] * 8))


def _mm0_body(x_ref, bfc_ref, d0_ref, d1_ref, w_ref, hh_ref, dinv_ref):
    dinv = lax.rsqrt(d0_ref[...] + d1_ref[...] + 1.0)
    x = jnp.maximum(x_ref[...] + bfc_ref[...], 0.0)
    h = lax.dot_general(x, w_ref[...],
                        (((1,), (1,)), ((), ())),
                        preferred_element_type=jnp.float32)
    hh_ref[...] = h * dinv
    dinv_ref[...] = dinv


def _mm0(x0, bfc2, d0, d1, W0):
    fout = W0.shape[0]
    return pl.pallas_call(
        _mm0_body,
        grid=(N // _ROW_BLK,),
        in_specs=[
            pl.BlockSpec((_ROW_BLK, LAT), lambda i: (i, 0)),
            pl.BlockSpec((_ROW_BLK, LAT), lambda i: (i, 0)),
            pl.BlockSpec((_ROW_BLK, 1), lambda i: (i, 0)),
            pl.BlockSpec((_ROW_BLK, 1), lambda i: (i, 0)),
            pl.BlockSpec((fout, LAT), lambda i: (0, 0)),
        ],
        out_specs=[
            pl.BlockSpec((_ROW_BLK, fout), lambda i: (i, 0)),
            pl.BlockSpec((_ROW_BLK, 1), lambda i: (i, 0)),
        ],
        out_shape=[
            jax.ShapeDtypeStruct((N, fout), jnp.float32),
            jax.ShapeDtypeStruct((N, 1), jnp.float32),
        ],
    )(x0, bfc2, d0, d1, W0)


def _mm_body(p0_ref, p1_ref, hh_ref, b_ref, dinv_ref, w_ref, o_ref):
    dinv = dinv_ref[...]
    x = jnp.maximum(dinv * (p0_ref[0] + p1_ref[0] + hh_ref[...])
                    + b_ref[...], 0.0)
    h = lax.dot_general(x, w_ref[...], (((1,), (1,)), ((), ())),
                        preferred_element_type=jnp.float32)
    o_ref[...] = h * dinv


def _mm(p, hh, b, dinv, W):
    fin = W.shape[1]
    fout = W.shape[0]
    return pl.pallas_call(
        _mm_body,
        grid=(N // _ROW_BLK,),
        in_specs=[
            pl.BlockSpec((1, _ROW_BLK, fin), lambda i: (0, i, 0)),
            pl.BlockSpec((1, _ROW_BLK, fin), lambda i: (1, i, 0)),
            pl.BlockSpec((_ROW_BLK, fin), lambda i: (i, 0)),
            pl.BlockSpec((1, fin), lambda i: (0, 0)),
            pl.BlockSpec((_ROW_BLK, 1), lambda i: (i, 0)),
            pl.BlockSpec((fout, fin), lambda i: (0, 0)),
        ],
        out_specs=pl.BlockSpec((_ROW_BLK, fout), lambda i: (i, 0)),
        out_shape=jax.ShapeDtypeStruct((N, fout), jnp.float32),
    )(p, p, hh, b.reshape(1, fin), dinv, W)


def _ep_body(p0_ref, p1_ref, hh_ref, b_ref, dinv_ref, o_ref):
    o_ref[...] = dinv_ref[...] * (p0_ref[0] + p1_ref[0] + hh_ref[...]) \
        + b_ref[...]


def _ep(p, hh, b, dinv):
    return pl.pallas_call(
        _ep_body,
        grid=(N // _ROW_BLK,),
        in_specs=[
            pl.BlockSpec((1, _ROW_BLK, NF), lambda i: (0, i, 0)),
            pl.BlockSpec((1, _ROW_BLK, NF), lambda i: (1, i, 0)),
            pl.BlockSpec((_ROW_BLK, NF), lambda i: (i, 0)),
            pl.BlockSpec((1, NF), lambda i: (0, 0)),
            pl.BlockSpec((_ROW_BLK, 1), lambda i: (i, 0)),
        ],
        out_specs=pl.BlockSpec((_ROW_BLK, NF), lambda i: (i, 0)),
        out_shape=jax.ShapeDtypeStruct((N, NF), jnp.float32),
    )(p, p, hh, b.reshape(1, NF), dinv)


def kernel(z, edge_index, Wfc, bfc, W0, b0, W1, b1, W2, b2, W3, b3, W4, b4, W5, b5):
    Ws = [W0, W1, W2, W3, W4, W5]
    bs = [b0, b1, b2, b3, b4, b5]
    edge = edge_index.astype(jnp.int32)

    deg = _deg_sc(edge)
    x0 = _fc(Wfc.T, z).reshape(N, LAT)
    d0 = deg[0, :N].reshape(N, 1)
    d1 = deg[1, :N].reshape(N, 1)

    hh, dinv = _mm0(x0, bfc.reshape(N, LAT), d0, d1, Ws[0])
    for i in range(1, N_LAYERS):
        p = _scatter_sc(edge, hh)
        hh = _mm(p, hh, bs[i - 1], dinv, Ws[i])
    p = _scatter_sc(edge, hh)
    return _ep(p, hh, bs[N_LAYERS - 1], dinv)
